# baseline (speedup 1.0000x reference)
import functools

import jax
import jax.numpy as jnp
from jax import lax
from jax.experimental import pallas as pl
from jax.experimental.pallas import tpu as pltpu
from jax.experimental.pallas import tpu_sc as plsc

N_NODES = 10000
N_EDGES = 320000
D = 128

_info = plsc.get_sparse_core_info()
_NC = _info.num_cores
_NW = _info.num_cores * _info.num_subcores
_EPW = N_EDGES // _NW
_C = 80
_NCHUNK = _EPW // _C


def _lin1_body(x_ref, w_ref, b_ref, o_ref):
    o_ref[...] = (
        jnp.dot(x_ref[...], w_ref[...], preferred_element_type=jnp.float32)
        + b_ref[...]
    )


def _lin1(x, w1t, b1):
    m = x.shape[0]
    bm = 1000
    return pl.pallas_call(
        _lin1_body,
        grid=(m // bm,),
        in_specs=[
            pl.BlockSpec((bm, D), lambda i: (i, 0)),
            pl.BlockSpec((D, D), lambda i: (0, 0)),
            pl.BlockSpec((1, D), lambda i: (0, 0)),
        ],
        out_specs=pl.BlockSpec((bm, D), lambda i: (i, 0)),
        out_shape=jax.ShapeDtypeStruct((m, D), jnp.float32),
    )(x, w1t, b1.reshape(1, D))


def _pack_word(a_j, a_i, b_j, b_i):
    va = jnp.maximum(a_j + a_i, 0.0)
    vb = jnp.maximum(b_j + b_i, 0.0)
    ba = lax.bitcast_convert_type(va, jnp.int32)
    bb = lax.bitcast_convert_type(vb, jnp.int32)
    ra = ba + 0x7FFF + (lax.shift_right_logical(ba, 16) & 1)
    rb = bb + 0x7FFF + (lax.shift_right_logical(bb, 16) & 1)
    return lax.shift_right_logical(ra, 16) | (rb & jnp.int32(-65536))


def _sc_body(h_hbm, src_hbm, dst_hbm, out_hbm,
             idx_src, idx_dst, rj0, rj1, ri0, ri1, ob0, ob1,
             sj0, sj1, si0, si1, so0, so1):
    wid = lax.axis_index("s") * _NC + lax.axis_index("c")
    base0 = wid * _EPW
    obase0 = wid * (_EPW // 2)
    rj = (rj0, rj1)
    ri = (ri0, ri1)
    ob = (ob0, ob1)
    sj = (sj0, sj1)
    si = (si0, si1)
    so = (so0, so1)

    pltpu.sync_copy(src_hbm.at[wid], idx_src)
    pltpu.sync_copy(dst_hbm.at[wid], idx_dst)

    def gathers(t, b):
        pltpu.async_copy(h_hbm.at[idx_src.at[t]], rj[b], sj[b])
        pltpu.async_copy(h_hbm.at[idx_dst.at[t]], ri[b], si[b])

    gathers(0, 0)
    gathers(1, 1)

    def half(t, b, first, issue_next):
        pltpu.make_async_copy(h_hbm.at[idx_src.at[t]], rj[b], sj[b]).wait()
        pltpu.make_async_copy(h_hbm.at[idx_dst.at[t]], ri[b], si[b]).wait()

        @pl.when(jnp.logical_not(first))
        def _():
            pltpu.make_async_copy(
                ob[b], out_hbm.at[pl.ds(obase0, _C // 2)], so[b]).wait()

        def row_body(m, c2):
            for e_off, col in ((0, 0), (1, 64)):
                r = m * 2 + e_off
                for q in range(4):
                    sa = pl.ds(q * 32, 16)
                    sb = pl.ds(q * 32 + 16, 16)
                    word = _pack_word(rj[b][r, sa], ri[b][r, sa],
                                      rj[b][r, sb], ri[b][r, sb])
                    ob[b][m, pl.ds(col + q * 16, 16)] = word
            return c2

        lax.fori_loop(0, _C // 2, row_body, 0)
        pltpu.async_copy(
            ob[b], out_hbm.at[pl.ds(obase0 + t * (_C // 2), _C // 2)], so[b])

        @pl.when(issue_next)
        def _():
            gathers(t + 2, b)

    def body(tt, carry):
        t = tt * 2
        half(t, 0, tt < 1, t + 2 < _NCHUNK)
        half(t + 1, 1, tt < 1, t + 3 < _NCHUNK)
        return carry

    lax.fori_loop(0, _NCHUNK // 2, body, 0)
    half(jnp.int32(_NCHUNK - 1), 0, jnp.bool_(False), jnp.bool_(False))

    for b in range(2):
        pltpu.make_async_copy(
            ob[b], out_hbm.at[pl.ds(obase0, _C // 2)], so[b]).wait()


def _gather_relu(h, src3, dst3):
    mesh = plsc.VectorSubcoreMesh(core_axis_name="c", subcore_axis_name="s")
    k = functools.partial(
        pl.kernel,
        out_type=jax.ShapeDtypeStruct((N_EDGES // 2, D), jnp.int32),
        mesh=mesh,
        scratch_types=[
            pltpu.VMEM((_NCHUNK, _C), jnp.int32),
            pltpu.VMEM((_NCHUNK, _C), jnp.int32),
            pltpu.VMEM((_C, D), jnp.float32),
            pltpu.VMEM((_C, D), jnp.float32),
            pltpu.VMEM((_C, D), jnp.float32),
            pltpu.VMEM((_C, D), jnp.float32),
            pltpu.VMEM((_C // 2, D), jnp.int32),
            pltpu.VMEM((_C // 2, D), jnp.int32),
            pltpu.SemaphoreType.DMA,
            pltpu.SemaphoreType.DMA,
            pltpu.SemaphoreType.DMA,
            pltpu.SemaphoreType.DMA,
            pltpu.SemaphoreType.DMA,
            pltpu.SemaphoreType.DMA,
        ],
    )(_sc_body)
    return k(h, src3, dst3)


def _lin2_body(xe_ref, ea_ref, ef_ref, wlo_ref, whi_ref, wb_ref, wc_ref,
               b_ref, o_ref):
    w = xe_ref[...]
    lo = lax.bitcast_convert_type(lax.shift_left(w, 16), jnp.float32)
    hi = lax.bitcast_convert_type(w & jnp.int32(-65536), jnp.float32)
    pair = jnp.dot(lo, wlo_ref[...], preferred_element_type=jnp.float32)
    pair = pair + jnp.dot(hi, whi_ref[...], preferred_element_type=jnp.float32)
    acc = pair.reshape(2 * pair.shape[0], D)
    acc = acc + jnp.dot(ea_ref[...], wb_ref[...],
                        preferred_element_type=jnp.float32)
    acc = acc + jnp.dot(ef_ref[...], wc_ref[...],
                        preferred_element_type=jnp.float32)
    o_ref[...] = acc + b_ref[...]


def _lin2(x_em, edge_attr, edge_f, wlo, whi, wb, wc, b2):
    e = edge_attr.shape[0]
    be = 2000
    ein = edge_attr.shape[1]
    nef = edge_f.shape[1]
    return pl.pallas_call(
        _lin2_body,
        grid=(e // be,),
        in_specs=[
            pl.BlockSpec((be // 2, D), lambda i: (i, 0)),
            pl.BlockSpec((be, ein), lambda i: (i, 0)),
            pl.BlockSpec((be, nef), lambda i: (i, 0)),
            pl.BlockSpec((D, 2 * D), lambda i: (0, 0)),
            pl.BlockSpec((D, 2 * D), lambda i: (0, 0)),
            pl.BlockSpec((ein, D), lambda i: (0, 0)),
            pl.BlockSpec((nef, D), lambda i: (0, 0)),
            pl.BlockSpec((1, D), lambda i: (0, 0)),
        ],
        out_specs=pl.BlockSpec((be, D), lambda i: (i, 0)),
        out_shape=jax.ShapeDtypeStruct((e, D), jnp.float32),
    )(x_em, edge_attr, edge_f, wlo, whi, wb, wc, b2.reshape(1, D))


def kernel(x, edge_index, edge_f, edge_attr, device, W1, b1, W2, b2):
    src = edge_index[0].astype(jnp.int32)
    dst = edge_index[1].astype(jnp.int32)
    h = _lin1(x, W1.T, b1)
    x_em = _gather_relu(h, src.reshape(_NW, _NCHUNK, _C),
                        dst.reshape(_NW, _NCHUNK, _C))
    w2t = W2.T
    ein = edge_attr.shape[1]
    k64 = jnp.arange(D // 2)
    feat = (k64 // 16) * 32 + (k64 % 16)
    z = jnp.zeros((D // 2, D), jnp.float32)
    wlo = jnp.concatenate(
        [jnp.concatenate([w2t[:D][feat], z], axis=1),
         jnp.concatenate([z, w2t[:D][feat]], axis=1)], axis=0)
    whi = jnp.concatenate(
        [jnp.concatenate([w2t[:D][feat + 16], z], axis=1),
         jnp.concatenate([z, w2t[:D][feat + 16]], axis=1)], axis=0)
    wb = w2t[D:D + ein]
    wc = w2t[D + ein:]
    return _lin2(x_em, edge_attr, edge_f, wlo, whi, wb, wc, b2)


# speedup vs baseline: 1.2237x; 1.2237x over previous
import functools

import jax
import jax.numpy as jnp
from jax import lax
from jax.experimental import pallas as pl
from jax.experimental.pallas import tpu as pltpu
from jax.experimental.pallas import tpu_sc as plsc

N_NODES = 10000
N_EDGES = 320000
D = 128

_info = plsc.get_sparse_core_info()
_NC = _info.num_cores
_NW = _info.num_cores * _info.num_subcores
_EPW = N_EDGES // _NW
_C = 80
_NCHUNK = _EPW // _C


def _lin1_body(x_ref, w_ref, b_ref, o_ref):
    o_ref[...] = (
        jnp.dot(x_ref[...], w_ref[...], preferred_element_type=jnp.float32)
        + b_ref[...]
    )


def _lin1(x, w1t, b1):
    m = x.shape[0]
    bm = 1000
    return pl.pallas_call(
        _lin1_body,
        grid=(m // bm,),
        in_specs=[
            pl.BlockSpec((bm, D), lambda i: (i, 0)),
            pl.BlockSpec((D, D), lambda i: (0, 0)),
            pl.BlockSpec((1, D), lambda i: (0, 0)),
        ],
        out_specs=pl.BlockSpec((bm, D), lambda i: (i, 0)),
        out_shape=jax.ShapeDtypeStruct((m, D), jnp.float32),
    )(x, w1t, b1.reshape(1, D))


def _sc_body(h_hbm, src_hbm, dst_hbm, out_hbm,
             idx_src, idx_dst, rj0, rj1, ri0, ri1, ob0, ob1,
             sj0, sj1, si0, si1, so0, so1):
    wid = lax.axis_index("s") * _NC + lax.axis_index("c")
    base0 = wid * _EPW
    rj = (rj0, rj1)
    ri = (ri0, ri1)
    ob = (ob0, ob1)
    sj = (sj0, sj1)
    si = (si0, si1)
    so = (so0, so1)

    pltpu.sync_copy(src_hbm.at[wid], idx_src)
    pltpu.sync_copy(dst_hbm.at[wid], idx_dst)

    def gathers(t, b):
        cj = pltpu.async_copy(h_hbm.at[idx_src.at[t]], rj[b], sj[b])
        ci = pltpu.async_copy(h_hbm.at[idx_dst.at[t]], ri[b], si[b])
        return cj, ci

    g0 = gathers(0, 0)
    g1 = gathers(1, 1)

    def half(t, b, first, issue_next):
        pltpu.make_async_copy(h_hbm.at[idx_src.at[t]], rj[b], sj[b]).wait()
        pltpu.make_async_copy(h_hbm.at[idx_dst.at[t]], ri[b], si[b]).wait()

        @pl.when(jnp.logical_not(first))
        def _():
            pltpu.make_async_copy(
                ob[b], out_hbm.at[pl.ds(base0, _C)], so[b]).wait()

        def row_body(r, c2):
            for u in range(D // 16):
                s = pl.ds(u * 16, 16)
                ob[b][r, s] = jnp.maximum(rj[b][r, s] + ri[b][r, s], 0.0)
            return c2

        lax.fori_loop(0, _C, row_body, 0)
        pltpu.async_copy(ob[b], out_hbm.at[pl.ds(base0 + t * _C, _C)], so[b])

        @pl.when(issue_next)
        def _():
            gathers(t + 2, b)

    def body(tt, carry):
        t = tt * 2
        half(t, 0, tt < 1, t + 2 < _NCHUNK)
        half(t + 1, 1, tt < 1, t + 3 < _NCHUNK)
        return carry

    lax.fori_loop(0, _NCHUNK // 2, body, 0)
    half(jnp.int32(_NCHUNK - 1), 0, jnp.bool_(False), jnp.bool_(False))

    for b in range(2):
        pltpu.make_async_copy(
            ob[b], out_hbm.at[pl.ds(base0, _C)], so[b]).wait()
    del g0, g1


def _gather_relu(h, src, dst):
    mesh = plsc.VectorSubcoreMesh(core_axis_name="c", subcore_axis_name="s")
    k = functools.partial(
        pl.kernel,
        out_type=jax.ShapeDtypeStruct((N_EDGES, D), jnp.float32),
        mesh=mesh,
        scratch_types=[
            pltpu.VMEM((_NCHUNK, _C), jnp.int32),
            pltpu.VMEM((_NCHUNK, _C), jnp.int32),
            pltpu.VMEM((_C, D), jnp.float32),
            pltpu.VMEM((_C, D), jnp.float32),
            pltpu.VMEM((_C, D), jnp.float32),
            pltpu.VMEM((_C, D), jnp.float32),
            pltpu.VMEM((_C, D), jnp.float32),
            pltpu.VMEM((_C, D), jnp.float32),
            pltpu.SemaphoreType.DMA,
            pltpu.SemaphoreType.DMA,
            pltpu.SemaphoreType.DMA,
            pltpu.SemaphoreType.DMA,
            pltpu.SemaphoreType.DMA,
            pltpu.SemaphoreType.DMA,
        ],
    )(_sc_body)
    return k(h, src.reshape(_NW, _NCHUNK, _C), dst.reshape(_NW, _NCHUNK, _C))


def _lin2_body(xe_ref, ea_ref, ef_ref, wa_ref, wb_ref, wc_ref, b_ref, o_ref):
    acc = jnp.dot(xe_ref[...], wa_ref[...], preferred_element_type=jnp.float32)
    acc = acc + jnp.dot(ea_ref[...], wb_ref[...],
                        preferred_element_type=jnp.float32)
    acc = acc + jnp.dot(ef_ref[...], wc_ref[...],
                        preferred_element_type=jnp.float32)
    o_ref[...] = acc + b_ref[...]


def _lin2(x_em, edge_attr, edge_f, wa, wb, wc, b2):
    e = x_em.shape[0]
    be = 2000
    ein = edge_attr.shape[1]
    ef = edge_f.shape[1]
    return pl.pallas_call(
        _lin2_body,
        grid=(e // be,),
        in_specs=[
            pl.BlockSpec((be, D), lambda i: (i, 0)),
            pl.BlockSpec((be, ein), lambda i: (i, 0)),
            pl.BlockSpec((be, ef), lambda i: (i, 0)),
            pl.BlockSpec((D, D), lambda i: (0, 0)),
            pl.BlockSpec((ein, D), lambda i: (0, 0)),
            pl.BlockSpec((ef, D), lambda i: (0, 0)),
            pl.BlockSpec((1, D), lambda i: (0, 0)),
        ],
        out_specs=pl.BlockSpec((be, D), lambda i: (i, 0)),
        out_shape=jax.ShapeDtypeStruct((e, D), jnp.float32),
    )(x_em, edge_attr, edge_f, wa, wb, wc, b2.reshape(1, D))


def kernel(x, edge_index, edge_f, edge_attr, device, W1, b1, W2, b2):
    src = edge_index[0].astype(jnp.int32)
    dst = edge_index[1].astype(jnp.int32)
    h = _lin1(x, W1.T, b1)
    x_em = _gather_relu(h, src, dst)
    w2t = W2.T
    ein = edge_attr.shape[1]
    wa = w2t[:D]
    wb = w2t[D:D + ein]
    wc = w2t[D + ein:]
    return _lin2(x_em, edge_attr, edge_f, wa, wb, wc, b2)


# speedup vs baseline: 1.3066x; 1.0677x over previous
import functools

import jax
import jax.numpy as jnp
from jax import lax
from jax.experimental import pallas as pl
from jax.experimental.pallas import tpu as pltpu
from jax.experimental.pallas import tpu_sc as plsc

N_NODES = 10000
N_EDGES = 320000
D = 128

_info = plsc.get_sparse_core_info()
_NC = _info.num_cores
_NW = _info.num_cores * _info.num_subcores
_EPW = N_EDGES // _NW
_C = 80
_NCHUNK = _EPW // _C


def _lin1_body(x_ref, w_ref, b_ref, o_ref):
    o_ref[...] = (
        jnp.dot(x_ref[...], w_ref[...], preferred_element_type=jnp.float32)
        + b_ref[...]
    )


def _lin1(x, w1t, b1):
    m = x.shape[0]
    bm = 1000
    return pl.pallas_call(
        _lin1_body,
        grid=(m // bm,),
        in_specs=[
            pl.BlockSpec((bm, D), lambda i: (i, 0)),
            pl.BlockSpec((D, D), lambda i: (0, 0)),
            pl.BlockSpec((1, D), lambda i: (0, 0)),
        ],
        out_specs=pl.BlockSpec((bm, D), lambda i: (i, 0)),
        out_shape=jax.ShapeDtypeStruct((m, D), jnp.float32),
    )(x, w1t, b1.reshape(1, D))


def _sc_body(h_hbm, src_hbm, dst_hbm, out_hbm,
             idx_src, idx_dst, rj0, rj1, ri0, ri1, ob0, ob1,
             sj0, sj1, si0, si1, so0, so1):
    wid = lax.axis_index("s") * _NC + lax.axis_index("c")
    base0 = wid * _EPW
    rj = (rj0, rj1)
    ri = (ri0, ri1)
    ob = (ob0, ob1)
    sj = (sj0, sj1)
    si = (si0, si1)
    so = (so0, so1)

    pltpu.sync_copy(src_hbm.at[wid], idx_src)
    pltpu.sync_copy(dst_hbm.at[wid], idx_dst)

    def gathers(t, b):
        cj = pltpu.async_copy(h_hbm.at[idx_src.at[t]], rj[b], sj[b])
        ci = pltpu.async_copy(h_hbm.at[idx_dst.at[t]], ri[b], si[b])
        return cj, ci

    g0 = gathers(0, 0)
    g1 = gathers(1, 1)

    def half(t, b, first, issue_next):
        pltpu.make_async_copy(h_hbm.at[idx_src.at[t]], rj[b], sj[b]).wait()
        pltpu.make_async_copy(h_hbm.at[idx_dst.at[t]], ri[b], si[b]).wait()

        @pl.when(jnp.logical_not(first))
        def _():
            pltpu.make_async_copy(
                ob[b], out_hbm.at[pl.ds(base0, _C)], so[b]).wait()

        def row_body(r, c2):
            for u in range(D // 16):
                s = pl.ds(u * 16, 16)
                ob[b][r, s] = jnp.maximum(rj[b][r, s] + ri[b][r, s], 0.0)
            return c2

        lax.fori_loop(0, _C, row_body, 0)
        pltpu.async_copy(ob[b], out_hbm.at[pl.ds(base0 + t * _C, _C)], so[b])

        @pl.when(issue_next)
        def _():
            gathers(t + 2, b)

    def body(tt, carry):
        t = tt * 2
        half(t, 0, tt < 1, t + 2 < _NCHUNK)
        half(t + 1, 1, tt < 1, t + 3 < _NCHUNK)
        return carry

    lax.fori_loop(0, _NCHUNK // 2, body, 0)
    half(jnp.int32(_NCHUNK - 1), 0, jnp.bool_(False), jnp.bool_(False))

    for b in range(2):
        pltpu.make_async_copy(
            ob[b], out_hbm.at[pl.ds(base0, _C)], so[b]).wait()
    del g0, g1


def _gather_relu(h, src, dst):
    mesh = plsc.VectorSubcoreMesh(core_axis_name="c", subcore_axis_name="s")
    k = functools.partial(
        pl.kernel,
        out_type=jax.ShapeDtypeStruct((N_EDGES, D), jnp.float32),
        mesh=mesh,
        scratch_types=[
            pltpu.VMEM((_NCHUNK, _C), jnp.int32),
            pltpu.VMEM((_NCHUNK, _C), jnp.int32),
            pltpu.VMEM((_C, D), jnp.float32),
            pltpu.VMEM((_C, D), jnp.float32),
            pltpu.VMEM((_C, D), jnp.float32),
            pltpu.VMEM((_C, D), jnp.float32),
            pltpu.VMEM((_C, D), jnp.float32),
            pltpu.VMEM((_C, D), jnp.float32),
            pltpu.SemaphoreType.DMA,
            pltpu.SemaphoreType.DMA,
            pltpu.SemaphoreType.DMA,
            pltpu.SemaphoreType.DMA,
            pltpu.SemaphoreType.DMA,
            pltpu.SemaphoreType.DMA,
        ],
    )(_sc_body)
    return k(h, src.reshape(_NW, _NCHUNK, _C), dst.reshape(_NW, _NCHUNK, _C))


def _lin2_body(xe_ref, ea_ref, ef_ref, wa_ref, wb_ref, wc_ref, b_ref, o_ref):
    acc = jnp.dot(xe_ref[...], wa_ref[...], preferred_element_type=jnp.float32)
    acc = acc + jnp.dot(ea_ref[...], wb_ref[...],
                        preferred_element_type=jnp.float32)
    acc = acc + jnp.dot(ef_ref[...], wc_ref[...],
                        preferred_element_type=jnp.float32)
    o_ref[...] = acc + b_ref[...]


def _lin2(x_em, edge_attr, edge_f, wa, wb, wc, b2):
    e = x_em.shape[0]
    be = 4000
    ein = edge_attr.shape[1]
    ef = edge_f.shape[1]
    return pl.pallas_call(
        _lin2_body,
        grid=(e // be,),
        in_specs=[
            pl.BlockSpec((be, D), lambda i: (i, 0)),
            pl.BlockSpec((be, ein), lambda i: (i, 0)),
            pl.BlockSpec((be, ef), lambda i: (i, 0)),
            pl.BlockSpec((D, D), lambda i: (0, 0)),
            pl.BlockSpec((ein, D), lambda i: (0, 0)),
            pl.BlockSpec((ef, D), lambda i: (0, 0)),
            pl.BlockSpec((1, D), lambda i: (0, 0)),
        ],
        out_specs=pl.BlockSpec((be, D), lambda i: (i, 0)),
        out_shape=jax.ShapeDtypeStruct((e, D), jnp.float32),
    )(x_em, edge_attr, edge_f, wa, wb, wc, b2.reshape(1, D))


def kernel(x, edge_index, edge_f, edge_attr, device, W1, b1, W2, b2):
    src = edge_index[0].astype(jnp.int32)
    dst = edge_index[1].astype(jnp.int32)
    h = _lin1(x, W1.T, b1)
    x_em = _gather_relu(h, src, dst)
    w2t = W2.T
    ein = edge_attr.shape[1]
    wa = w2t[:D]
    wb = w2t[D:D + ein]
    wc = w2t[D + ein:]
    return _lin2(x_em, edge_attr, edge_f, wa, wb, wc, b2)


# speedup vs baseline: 1.3198x; 1.0101x over previous
import functools

import jax
import jax.numpy as jnp
from jax import lax
from jax.experimental import pallas as pl
from jax.experimental.pallas import tpu as pltpu
from jax.experimental.pallas import tpu_sc as plsc

N_NODES = 10000
N_EDGES = 320000
D = 128

_info = plsc.get_sparse_core_info()
_NC = _info.num_cores
_NW = _info.num_cores * _info.num_subcores
_EPW = N_EDGES // _NW
_C = 80
_NCHUNK = _EPW // _C


def _lin1_body(x_ref, w_ref, b_ref, o_ref):
    o_ref[...] = (
        jnp.dot(x_ref[...], w_ref[...], preferred_element_type=jnp.float32)
        + b_ref[...]
    )


def _lin1(x, w1t, b1):
    m = x.shape[0]
    bm = 1000
    return pl.pallas_call(
        _lin1_body,
        grid=(m // bm,),
        in_specs=[
            pl.BlockSpec((bm, D), lambda i: (i, 0)),
            pl.BlockSpec((D, D), lambda i: (0, 0)),
            pl.BlockSpec((1, D), lambda i: (0, 0)),
        ],
        out_specs=pl.BlockSpec((bm, D), lambda i: (i, 0)),
        out_shape=jax.ShapeDtypeStruct((m, D), jnp.float32),
    )(x, w1t, b1.reshape(1, D))


def _sc_body(h_hbm, src_hbm, dst_hbm, out_hbm,
             idx_src, idx_dst, rj0, rj1, ri0, ri1, ob0, ob1,
             sj0, sj1, si0, si1, so0, so1):
    wid = lax.axis_index("s") * _NC + lax.axis_index("c")
    base0 = wid * _EPW
    rj = (rj0, rj1)
    ri = (ri0, ri1)
    ob = (ob0, ob1)
    sj = (sj0, sj1)
    si = (si0, si1)
    so = (so0, so1)

    pltpu.sync_copy(src_hbm.at[wid], idx_src)
    pltpu.sync_copy(dst_hbm.at[wid], idx_dst)

    def gathers(t, b):
        cj = pltpu.async_copy(h_hbm.at[idx_src.at[t]], rj[b], sj[b])
        ci = pltpu.async_copy(h_hbm.at[idx_dst.at[t]], ri[b], si[b])
        return cj, ci

    g0 = gathers(0, 0)
    g1 = gathers(1, 1)

    def half(t, b, first, issue_next):
        pltpu.make_async_copy(h_hbm.at[idx_src.at[t]], rj[b], sj[b]).wait()
        pltpu.make_async_copy(h_hbm.at[idx_dst.at[t]], ri[b], si[b]).wait()

        @pl.when(jnp.logical_not(first))
        def _():
            pltpu.make_async_copy(
                ob[b], out_hbm.at[pl.ds(base0, _C)], so[b]).wait()

        def row_body(r, c2):
            for u in range(D // 16):
                s = pl.ds(u * 16, 16)
                ob[b][r, s] = jnp.maximum(rj[b][r, s] + ri[b][r, s], 0.0)
            return c2

        lax.fori_loop(0, _C, row_body, 0)
        pltpu.async_copy(ob[b], out_hbm.at[pl.ds(base0 + t * _C, _C)], so[b])

        @pl.when(issue_next)
        def _():
            gathers(t + 2, b)

    def body(tt, carry):
        t = tt * 2
        half(t, 0, tt < 1, t + 2 < _NCHUNK)
        half(t + 1, 1, tt < 1, t + 3 < _NCHUNK)
        return carry

    lax.fori_loop(0, _NCHUNK // 2, body, 0)
    half(jnp.int32(_NCHUNK - 1), 0, jnp.bool_(False), jnp.bool_(False))

    for b in range(2):
        pltpu.make_async_copy(
            ob[b], out_hbm.at[pl.ds(base0, _C)], so[b]).wait()
    del g0, g1


def _gather_relu(h, src, dst):
    mesh = plsc.VectorSubcoreMesh(core_axis_name="c", subcore_axis_name="s")
    k = functools.partial(
        pl.kernel,
        out_type=jax.ShapeDtypeStruct((N_EDGES, D), jnp.float32),
        mesh=mesh,
        scratch_types=[
            pltpu.VMEM((_NCHUNK, _C), jnp.int32),
            pltpu.VMEM((_NCHUNK, _C), jnp.int32),
            pltpu.VMEM((_C, D), jnp.float32),
            pltpu.VMEM((_C, D), jnp.float32),
            pltpu.VMEM((_C, D), jnp.float32),
            pltpu.VMEM((_C, D), jnp.float32),
            pltpu.VMEM((_C, D), jnp.float32),
            pltpu.VMEM((_C, D), jnp.float32),
            pltpu.SemaphoreType.DMA,
            pltpu.SemaphoreType.DMA,
            pltpu.SemaphoreType.DMA,
            pltpu.SemaphoreType.DMA,
            pltpu.SemaphoreType.DMA,
            pltpu.SemaphoreType.DMA,
        ],
    )(_sc_body)
    return k(h, src.reshape(_NW, _NCHUNK, _C), dst.reshape(_NW, _NCHUNK, _C))


def _lin2_body(xe_ref, ea_ref, ef_ref, wa_ref, wb_ref, wc_ref, b_ref, o_ref):
    acc = jnp.dot(xe_ref[...], wa_ref[...], preferred_element_type=jnp.float32)
    acc = acc + jnp.dot(ea_ref[...], wb_ref[...],
                        preferred_element_type=jnp.float32)
    acc = acc + jnp.dot(ef_ref[...], wc_ref[...],
                        preferred_element_type=jnp.float32)
    o_ref[...] = acc + b_ref[...]


def _lin2(x_em, edge_attr, edge_f, wa, wb, wc, b2):
    e = x_em.shape[0]
    be = 8000
    ein = edge_attr.shape[1]
    ef = edge_f.shape[1]
    return pl.pallas_call(
        _lin2_body,
        grid=(e // be,),
        in_specs=[
            pl.BlockSpec((be, D), lambda i: (i, 0)),
            pl.BlockSpec((be, ein), lambda i: (i, 0)),
            pl.BlockSpec((be, ef), lambda i: (i, 0)),
            pl.BlockSpec((D, D), lambda i: (0, 0)),
            pl.BlockSpec((ein, D), lambda i: (0, 0)),
            pl.BlockSpec((ef, D), lambda i: (0, 0)),
            pl.BlockSpec((1, D), lambda i: (0, 0)),
        ],
        out_specs=pl.BlockSpec((be, D), lambda i: (i, 0)),
        out_shape=jax.ShapeDtypeStruct((e, D), jnp.float32),
    )(x_em, edge_attr, edge_f, wa, wb, wc, b2.reshape(1, D))


def kernel(x, edge_index, edge_f, edge_attr, device, W1, b1, W2, b2):
    src = edge_index[0].astype(jnp.int32)
    dst = edge_index[1].astype(jnp.int32)
    h = _lin1(x, W1.T, b1)
    x_em = _gather_relu(h, src, dst)
    w2t = W2.T
    ein = edge_attr.shape[1]
    wa = w2t[:D]
    wb = w2t[D:D + ein]
    wc = w2t[D + ein:]
    return _lin2(x_em, edge_attr, edge_f, wa, wb, wc, b2)
